# baseline (speedup 1.0000x reference)
import functools

import jax
import jax.numpy as jnp
from jax import lax
from jax.experimental import pallas as pl
from jax.experimental.pallas import tpu as pltpu
from jax.experimental.pallas import tpu_sc as plsc


def _gather_kernel(n_rows, d, num_cores, num_workers, chunk, nbuf):
  per_w = n_rows // num_workers
  n_chunks = per_w // chunk
  groups = n_chunks // nbuf

  mesh = plsc.VectorSubcoreMesh(core_axis_name="c", subcore_axis_name="s")

  scratch = (
      [pltpu.VMEM((n_chunks, chunk), jnp.int32)]
      + [pltpu.VMEM((chunk, d), jnp.float32) for _ in range(nbuf)]
      + [pltpu.SemaphoreType.DMA for _ in range(2 * nbuf + 1)]
  )

  @functools.partial(
      pl.kernel,
      out_type=jax.ShapeDtypeStruct((n_rows, d), jnp.float32),
      mesh=mesh,
      scratch_types=scratch,
      compiler_params=pltpu.CompilerParams(use_tc_tiling_on_sc=False),
  )
  def run(table, idx_hbm, out, idx_v, *rest):
    bufs = rest[:nbuf]
    gsem = rest[nbuf:2 * nbuf]
    osem = rest[2 * nbuf:3 * nbuf]
    isem = rest[3 * nbuf]

    wid = lax.axis_index("s") * num_cores + lax.axis_index("c")
    base = wid * per_w

    pltpu.async_copy(idx_hbm.at[wid], idx_v, isem).wait()

    def start_gather(j, b):
      pltpu.async_copy(table.at[idx_v.at[j]], bufs[b], gsem[b])

    def wait_gather(b):
      pltpu.make_async_copy(
          out.at[pl.ds(base, chunk)], bufs[b], gsem[b]).wait()

    def start_out(j, b):
      pltpu.async_copy(bufs[b], out.at[pl.ds(base + j * chunk, chunk)],
                       osem[b])

    def wait_out(b):
      pltpu.make_async_copy(
          bufs[b], out.at[pl.ds(base, chunk)], osem[b]).wait()

    for b in range(nbuf):
      start_gather(b, b)

    def group_body(g, carry):
      for b in range(nbuf):
        j = g * nbuf + b
        wait_gather(b)
        start_out(j, b)
      for b in range(nbuf):
        wait_out(b)
        start_gather((g + 1) * nbuf + b, b)
      return carry

    lax.fori_loop(0, groups - 1, group_body, 0)

    for b in range(nbuf):
      j = (groups - 1) * nbuf + b
      wait_gather(b)
      start_out(j, b)
    for b in range(nbuf):
      wait_out(b)

  return run


def kernel(inlets, weight):
  b, s = inlets.shape
  v, d = weight.shape
  n = b * s

  info = plsc.get_sparse_core_info()
  num_workers = info.num_cores * info.num_subcores
  chunk = 128
  nbuf = 4

  idx = inlets.astype(jnp.int32).reshape(
      num_workers, n // (num_workers * chunk), chunk)
  run = _gather_kernel(n, d, info.num_cores, num_workers, chunk, nbuf)
  out = run(weight, idx)
  return out.reshape(b, s, d)


# speedup vs baseline: 1.2113x; 1.2113x over previous
import functools

import jax
import jax.numpy as jnp
from jax import lax
from jax.experimental import pallas as pl
from jax.experimental.pallas import tpu as pltpu
from jax.experimental.pallas import tpu_sc as plsc


def _gather_kernel(n_rows, num_cores, num_workers, seq, nbuf):
  bpw = n_rows // (num_workers * seq)
  half = seq // 2
  groups = bpw // nbuf

  mesh = plsc.VectorSubcoreMesh(core_axis_name="c", subcore_axis_name="s")

  scratch = (
      [pltpu.VMEM((bpw, 2, half), jnp.int32)]
      + [pltpu.VMEM((seq, 128), jnp.float32) for _ in range(nbuf)]
      + [pltpu.SemaphoreType.DMA for _ in range(2 * nbuf + 1)]
  )

  @functools.partial(
      pl.kernel,
      out_type=jax.ShapeDtypeStruct((n_rows, 128), jnp.float32),
      mesh=mesh,
      scratch_types=scratch,
      compiler_params=pltpu.CompilerParams(use_tc_tiling_on_sc=True),
  )
  def run(table, idx_hbm, out, idx_v, *rest):
    bufs = rest[:nbuf]
    gsem = rest[nbuf:2 * nbuf]
    osem = rest[2 * nbuf:3 * nbuf]
    isem = rest[3 * nbuf]

    wid = lax.axis_index("s") * num_cores + lax.axis_index("c")
    base = wid * bpw * seq

    pltpu.async_copy(idx_hbm.at[wid], idx_v, isem).wait()

    def start_gather(k, b):
      pltpu.async_copy(table.at[idx_v.at[k, 0]], bufs[b].at[pl.ds(0, half)],
                       gsem[b])
      pltpu.async_copy(table.at[idx_v.at[k, 1]],
                       bufs[b].at[pl.ds(half, half)], gsem[b])

    def wait_gather(b):
      pltpu.make_async_copy(out.at[pl.ds(base, seq)], bufs[b], gsem[b]).wait()

    def start_out(k, b):
      pltpu.async_copy(bufs[b], out.at[pl.ds(base + k * seq, seq)], osem[b])

    def wait_out(b):
      pltpu.make_async_copy(bufs[b], out.at[pl.ds(base, seq)], osem[b]).wait()

    for b in range(nbuf):
      start_gather(b, b)

    def group_body(g, carry):
      for b in range(nbuf):
        wait_gather(b)
        start_out(g * nbuf + b, b)
      for b in range(nbuf):
        wait_out(b)
        start_gather((g + 1) * nbuf + b, b)
      return carry

    lax.fori_loop(0, groups - 1, group_body, 0)

    for b in range(nbuf):
      wait_gather(b)
      start_out((groups - 1) * nbuf + b, b)
    for b in range(nbuf):
      wait_out(b)

  return run


def kernel(inlets, weight):
  b, s = inlets.shape
  v, d = weight.shape
  n = b * s

  info = plsc.get_sparse_core_info()
  num_workers = info.num_cores * info.num_subcores

  w128 = jnp.pad(weight, ((0, 0), (0, 128 - d)))
  idx = inlets.astype(jnp.int32).reshape(num_workers, b // num_workers, 2,
                                         s // 2)
  run = _gather_kernel(n, info.num_cores, num_workers, s, 2)
  out = run(w128, idx)
  return out[:, :d].reshape(b, s, d)


# speedup vs baseline: 1.2161x; 1.0040x over previous
import functools

import jax
import jax.numpy as jnp
from jax import lax
from jax.experimental import pallas as pl
from jax.experimental.pallas import tpu as pltpu
from jax.experimental.pallas import tpu_sc as plsc


def _gather_kernel(n_rows, num_cores, num_workers, seq, nbuf):
  per_w = n_rows // num_workers
  half = 128
  n_chunks = per_w // half
  groups = n_chunks // nbuf

  mesh = plsc.VectorSubcoreMesh(core_axis_name="c", subcore_axis_name="s")

  scratch = (
      [pltpu.VMEM((n_chunks, half), jnp.int32)]
      + [pltpu.VMEM((half, 128), jnp.float32) for _ in range(nbuf)]
      + [pltpu.SemaphoreType.DMA for _ in range(2 * nbuf + 1)]
  )

  @functools.partial(
      pl.kernel,
      out_type=jax.ShapeDtypeStruct((n_rows, 128), jnp.float32),
      mesh=mesh,
      scratch_types=scratch,
      compiler_params=pltpu.CompilerParams(use_tc_tiling_on_sc=True),
  )
  def run(table, idx_hbm, out, idx_v, *rest):
    bufs = rest[:nbuf]
    gsem = rest[nbuf:2 * nbuf]
    osem = rest[2 * nbuf:3 * nbuf]
    isem = rest[3 * nbuf]

    wid = lax.axis_index("s") * num_cores + lax.axis_index("c")
    base = wid * per_w

    pltpu.async_copy(idx_hbm.at[wid], idx_v, isem).wait()

    def start_gather(k, b):
      pltpu.async_copy(table.at[idx_v.at[k]], bufs[b], gsem[b])

    def wait_gather(b):
      pltpu.make_async_copy(out.at[pl.ds(base, half)], bufs[b], gsem[b]).wait()

    def start_out(k, b):
      pltpu.async_copy(bufs[b], out.at[pl.ds(base + k * half, half)], osem[b])

    def wait_out(b):
      pltpu.make_async_copy(bufs[b], out.at[pl.ds(base, half)], osem[b]).wait()

    for b in range(nbuf):
      start_gather(b, b)

    def group_body(g, carry):
      for b in range(nbuf):
        wait_gather(b)
        start_out(g * nbuf + b, b)
      for b in range(nbuf):
        wait_out(b)
        start_gather((g + 1) * nbuf + b, b)
      return carry

    lax.fori_loop(0, groups - 1, group_body, 0)

    for b in range(nbuf):
      wait_gather(b)
      start_out((groups - 1) * nbuf + b, b)
    for b in range(nbuf):
      wait_out(b)

  return run


def kernel(inlets, weight):
  b, s = inlets.shape
  v, d = weight.shape
  n = b * s

  info = plsc.get_sparse_core_info()
  num_workers = info.num_cores * info.num_subcores

  w128 = jnp.pad(weight, ((0, 0), (0, 128 - d)))
  idx = inlets.astype(jnp.int32).reshape(num_workers,
                                         n // (num_workers * 128), 128)
  run = _gather_kernel(n, info.num_cores, num_workers, s, 4)
  out = run(w128, idx)
  return out[:, :d].reshape(b, s, d)
